# baseline (speedup 1.0000x reference)
import functools

import jax
import jax.numpy as jnp
from jax import lax
from jax.experimental import pallas as pl
from jax.experimental.pallas import tpu as pltpu
from jax.experimental.pallas import tpu_sc as plsc

N = 50000
M = 4096
K = 8
D_ANCHOR = 512
D_LOCAL = 512
D_IN = D_ANCHOR + D_LOCAL
D_HID = 2048
D_OUT = 1024

NW = 32
CP = 12
PWH = 1632
NP = NW * PWH
CHUNKS = PWH // CP
NBUF = 2
SLACK = NBUF * CP * K
LANES = 16
NCOL = D_ANCHOR // LANES
BN = 400


def _take16(vec, idx):
    return lax.gather(
        vec, idx[:, None],
        dimension_numbers=lax.GatherDimensionNumbers(
            offset_dims=(), collapsed_slice_dims=(0,), start_index_map=(0,)),
        slice_sizes=(1,),
        mode=lax.GatherScatterMode.PROMISE_IN_BOUNDS)


def _seg_reduce(val, op):
    iota = lax.iota(jnp.int32, LANES)
    for s in (1, 2, 4):
        val = op(val, _take16(val, lax.bitwise_xor(iota, s)))
    return val


def _sc_pool_body(anchor_hbm, idxf_hbm, distf_hbm, tempv_hbm, out_hbm,
                  idx0_v, idx1_v, dist_v, rows0_v, rows1_v, pool_v, temp_v,
                  sem0, sem1):
    wid = lax.axis_index("s") * 2 + lax.axis_index("c")
    base_pt = wid * PWH
    pltpu.sync_copy(tempv_hbm, temp_v)
    tv = temp_v[pl.ds(0, LANES)]
    bufs = ((idx0_v, rows0_v, sem0), (idx1_v, rows1_v, sem1))

    for b in range(NBUF):
        idx_b, rows_b, sem_b = bufs[b]
        pltpu.sync_copy(idxf_hbm.at[pl.ds((base_pt + b * CP) * K, CP * K)],
                        idx_b)
        pltpu.async_copy(anchor_hbm.at[idx_b], rows_b, sem_b)

    def do_chunk(ci, pb, idx_b, rows_b, sem_b):
        pt0 = base_pt + ci * CP
        pltpu.sync_copy(distf_hbm.at[pl.ds(pt0 * K, CP * K)], dist_v)
        pltpu.make_async_copy(anchor_hbm.at[idx_b], rows_b, sem_b).wait()

        def pair_body(g, c2):
            d16 = dist_v[pl.ds(pl.multiple_of(g * LANES, LANES), LANES)]
            logits = -(tv * d16)
            m = _seg_reduce(logits, jnp.maximum)
            e = jnp.exp(logits - m)
            ssum = _seg_reduce(e, jnp.add)
            w16 = e / ssum
            row0 = g * LANES
            wb = [_take16(w16, jnp.full((LANES,), j, jnp.int32))
                  for j in range(LANES)]
            for half in range(2):
                p = pb * CP + g * 2 + half
                r0 = row0 + half * K
                for c in range(NCOL):
                    cs = pl.ds(c * LANES, LANES)
                    acc = wb[half * K] * rows_b[r0, cs]
                    for k in range(1, K):
                        acc = acc + wb[half * K + k] * rows_b[r0 + k, cs]
                    pool_v[p, cs] = acc
            return c2

        lax.fori_loop(0, CP * K // (2 * K), pair_body, 0)
        pltpu.sync_copy(idxf_hbm.at[pl.ds((pt0 + NBUF * CP) * K, CP * K)],
                        idx_b)
        pltpu.async_copy(anchor_hbm.at[idx_b], rows_b, sem_b)
        if pb == 1:
            pltpu.sync_copy(
                pool_v,
                out_hbm.at[pl.ds(pl.multiple_of(pt0 - CP, 2 * CP), 2 * CP)])

    def outer_body(i, carry):
        ci0 = i * NBUF
        for b in range(NBUF):
            idx_b, rows_b, sem_b = bufs[b]
            do_chunk(ci0 + b, b % 2, idx_b, rows_b, sem_b)
        return carry

    lax.fori_loop(0, CHUNKS // NBUF, outer_body, 0)
    for b in range(NBUF):
        idx_b, rows_b, sem_b = bufs[b]
        pltpu.make_async_copy(anchor_hbm.at[idx_b], rows_b, sem_b).wait()


_sc_pool = functools.partial(
    pl.kernel,
    out_type=jax.ShapeDtypeStruct((NP, D_ANCHOR), jnp.float32),
    mesh=plsc.VectorSubcoreMesh(core_axis_name="c", subcore_axis_name="s"),
    scratch_types=[
        pltpu.VMEM((CP * K,), jnp.int32),
        pltpu.VMEM((CP * K,), jnp.int32),
        pltpu.VMEM((CP * K,), jnp.float32),
        pltpu.VMEM((CP * K, D_ANCHOR), jnp.float32),
        pltpu.VMEM((CP * K, D_ANCHOR), jnp.float32),
        pltpu.VMEM((2 * CP, D_ANCHOR), jnp.float32),
        pltpu.VMEM((128,), jnp.float32),
        pltpu.SemaphoreType.DMA,
        pltpu.SemaphoreType.DMA,
    ],
)(_sc_pool_body)


def _tc_mlp_body(pooled_ref, local_ref, lns_ref, lnb_ref,
                 w1_ref, b1_ref, w2_ref, b2_ref, out_ref):
    comb = jnp.concatenate([pooled_ref[...], local_ref[...]], axis=1)
    mu = jnp.mean(comb, axis=1, keepdims=True)
    c = comb - mu
    var = jnp.mean(c * c, axis=1, keepdims=True)
    h = c * lax.rsqrt(var + 1e-5) * lns_ref[...] + lnb_ref[...]
    h1 = jnp.dot(h.astype(jnp.bfloat16), w1_ref[...],
                 preferred_element_type=jnp.float32) + b1_ref[...]
    g = 0.5 * h1 * (1.0 + lax.erf(h1 * 0.7071067811865476))
    out_ref[...] = (
        jnp.dot(g.astype(jnp.bfloat16), w2_ref[...],
                preferred_element_type=jnp.float32) + b2_ref[...])


def _tc_mlp(pooled, local_feats, ln_scale, ln_bias, W1, b1, W2, b2):
    return pl.pallas_call(
        _tc_mlp_body,
        grid=(N // BN,),
        in_specs=[
            pl.BlockSpec((BN, D_ANCHOR), lambda i: (i, 0)),
            pl.BlockSpec((BN, D_LOCAL), lambda i: (i, 0)),
            pl.BlockSpec((D_IN,), lambda i: (0,)),
            pl.BlockSpec((D_IN,), lambda i: (0,)),
            pl.BlockSpec((D_IN, D_HID), lambda i: (0, 0)),
            pl.BlockSpec((D_HID,), lambda i: (0,)),
            pl.BlockSpec((D_HID, D_OUT), lambda i: (0, 0)),
            pl.BlockSpec((D_OUT,), lambda i: (0,)),
        ],
        out_specs=pl.BlockSpec((BN, D_OUT), lambda i: (i, 0)),
        out_shape=jax.ShapeDtypeStruct((N, D_OUT), jnp.float32),
        compiler_params=pltpu.CompilerParams(
            dimension_semantics=("arbitrary",),
        ),
    )(pooled, local_feats, ln_scale, ln_bias, W1, b1, W2, b2)


def kernel(anchor_feats, local_feats, anchor_idx_per_point, dists,
           log_temp, ln_scale, ln_bias, W1, b1, W2, b2):
    idxf = jnp.pad(anchor_idx_per_point.astype(jnp.int32),
                   ((0, NP - N), (0, 0))).reshape(-1)
    idxf = jnp.concatenate([idxf, jnp.zeros((SLACK,), jnp.int32)])
    distf = jnp.pad(dists, ((0, NP - N), (0, 0))).reshape(-1)
    temp = jnp.clip(jnp.exp(log_temp), 0.5, 200.0).astype(jnp.float32)
    tempv = jnp.broadcast_to(temp, (128,))

    pooled = _sc_pool(anchor_feats, idxf, distf, tempv)
    return _tc_mlp(pooled, local_feats, ln_scale, ln_bias,
                   W1.astype(jnp.bfloat16), b1, W2.astype(jnp.bfloat16), b2)


# speedup vs baseline: 1.1671x; 1.1671x over previous
import functools

import jax
import jax.numpy as jnp
from jax import lax
from jax.experimental import pallas as pl
from jax.experimental.pallas import tpu as pltpu
from jax.experimental.pallas import tpu_sc as plsc

N = 50000
M = 4096
K = 8
D_ANCHOR = 512
D_LOCAL = 512
D_IN = D_ANCHOR + D_LOCAL
D_HID = 2048
D_OUT = 1024

NW = 32
CP = 8
H = 4
NP = 51200
PART = NP // H
PWH = PART // NW
CHUNKS = PWH // CP
NBUF = 2
SLACK = NBUF * CP * K
LANES = 16
NCOL = D_ANCHOR // LANES
BN = 400
BLK_PER_PART = PART // BN


def _take16(vec, idx):
    return lax.gather(
        vec, idx[:, None],
        dimension_numbers=lax.GatherDimensionNumbers(
            offset_dims=(), collapsed_slice_dims=(0,), start_index_map=(0,)),
        slice_sizes=(1,),
        mode=lax.GatherScatterMode.PROMISE_IN_BOUNDS)


def _seg_reduce(val, op):
    iota = lax.iota(jnp.int32, LANES)
    for s in (1, 2, 4):
        val = op(val, _take16(val, lax.bitwise_xor(iota, s)))
    return val


def _sc_pool_body(anchor_hbm, idxf_hbm, distf_hbm, tempv_hbm, out_hbm,
                  idx0_v, idx1_v, dist_v, rows0_v, rows1_v, pool_v, temp_v,
                  sem0, sem1, psem):
    wid = lax.axis_index("s") * 2 + lax.axis_index("c")
    base_pt = wid * PWH
    pltpu.sync_copy(tempv_hbm, temp_v)
    tv = temp_v[pl.ds(0, LANES)]
    bufs = ((idx0_v, rows0_v, sem0), (idx1_v, rows1_v, sem1))

    for b in range(NBUF):
        idx_b, rows_b, sem_b = bufs[b]
        pltpu.sync_copy(idxf_hbm.at[pl.ds((base_pt + b * CP) * K, CP * K)],
                        idx_b)
        pltpu.async_copy(anchor_hbm.at[idx_b], rows_b, sem_b)

    def do_chunk(ci, pb, idx_b, rows_b, sem_b):
        pt0 = base_pt + ci * CP
        pltpu.sync_copy(distf_hbm.at[pl.ds(pt0 * K, CP * K)], dist_v)
        pltpu.make_async_copy(anchor_hbm.at[idx_b], rows_b, sem_b).wait()
        if pb == 0:
            @pl.when(ci > 1)
            def _wait_flush():
                pltpu.make_async_copy(
                    pool_v,
                    out_hbm.at[pl.ds(
                        pl.multiple_of(pt0 - 2 * CP, 2 * CP), 2 * CP)],
                    psem).wait()

        def pair_body(g, c2):
            d16 = dist_v[pl.ds(pl.multiple_of(g * LANES, LANES), LANES)]
            logits = -(tv * d16)
            m = _seg_reduce(logits, jnp.maximum)
            e = jnp.exp(logits - m)
            ssum = _seg_reduce(e, jnp.add)
            w16 = e / ssum
            row0 = g * LANES
            wb = [_take16(w16, jnp.full((LANES,), j, jnp.int32))
                  for j in range(LANES)]
            for half in range(2):
                p = pb * CP + g * 2 + half
                r0 = row0 + half * K
                for c in range(NCOL):
                    cs = pl.ds(c * LANES, LANES)
                    acc = wb[half * K] * rows_b[r0, cs]
                    for k in range(1, K):
                        acc = acc + wb[half * K + k] * rows_b[r0 + k, cs]
                    pool_v[p, cs] = acc
            return c2

        lax.fori_loop(0, CP * K // (2 * K), pair_body, 0)
        pltpu.sync_copy(idxf_hbm.at[pl.ds((pt0 + NBUF * CP) * K, CP * K)],
                        idx_b)
        pltpu.async_copy(anchor_hbm.at[idx_b], rows_b, sem_b)
        if pb == 1:
            pltpu.async_copy(
                pool_v,
                out_hbm.at[pl.ds(pl.multiple_of(pt0 - CP, 2 * CP), 2 * CP)],
                psem)

    def outer_body(i, carry):
        ci0 = i * NBUF
        for b in range(NBUF):
            idx_b, rows_b, sem_b = bufs[b]
            do_chunk(ci0 + b, b % 2, idx_b, rows_b, sem_b)
        return carry

    lax.fori_loop(0, CHUNKS // NBUF, outer_body, 0)
    last_off = base_pt + (CHUNKS - 2) * CP
    pltpu.make_async_copy(
        pool_v,
        out_hbm.at[pl.ds(pl.multiple_of(last_off, 2 * CP), 2 * CP)],
        psem).wait()
    for b in range(NBUF):
        idx_b, rows_b, sem_b = bufs[b]
        pltpu.make_async_copy(anchor_hbm.at[idx_b], rows_b, sem_b).wait()


_sc_pool = functools.partial(
    pl.kernel,
    out_type=jax.ShapeDtypeStruct((PART, D_ANCHOR), jnp.float32),
    mesh=plsc.VectorSubcoreMesh(core_axis_name="c", subcore_axis_name="s"),
    scratch_types=[
        pltpu.VMEM((CP * K,), jnp.int32),
        pltpu.VMEM((CP * K,), jnp.int32),
        pltpu.VMEM((CP * K,), jnp.float32),
        pltpu.VMEM((CP * K, D_ANCHOR), jnp.float32),
        pltpu.VMEM((CP * K, D_ANCHOR), jnp.float32),
        pltpu.VMEM((2 * CP, D_ANCHOR), jnp.float32),
        pltpu.VMEM((128,), jnp.float32),
        pltpu.SemaphoreType.DMA,
        pltpu.SemaphoreType.DMA,
        pltpu.SemaphoreType.DMA,
    ],
)(_sc_pool_body)


def _tc_mlp_body(pooled_ref, local_ref, lns_ref, lnb_ref,
                 w1_ref, b1_ref, w2_ref, b2_ref, out_ref):
    comb = jnp.concatenate([pooled_ref[...], local_ref[...]], axis=1)
    mu = jnp.mean(comb, axis=1, keepdims=True)
    c = comb - mu
    var = jnp.mean(c * c, axis=1, keepdims=True)
    h = c * lax.rsqrt(var + 1e-5) * lns_ref[...] + lnb_ref[...]
    h1 = jnp.dot(h.astype(jnp.bfloat16), w1_ref[...],
                 preferred_element_type=jnp.float32) + b1_ref[...]
    g = 0.5 * h1 * (1.0 + lax.erf(h1 * 0.7071067811865476))
    out_ref[...] = (
        jnp.dot(g.astype(jnp.bfloat16), w2_ref[...],
                preferred_element_type=jnp.float32) + b2_ref[...])


def _tc_mlp_chain_body(pooled_ref, local_ref, lns_ref, lnb_ref,
                       w1_ref, b1_ref, w2_ref, b2_ref, carry_ref, out_ref):
    del carry_ref
    _tc_mlp_body(pooled_ref, local_ref, lns_ref, lnb_ref,
                 w1_ref, b1_ref, w2_ref, b2_ref, out_ref)


def _mk_specs(blk0):
    in_specs = [
        pl.BlockSpec((BN, D_ANCHOR), lambda i: (i, 0)),
        pl.BlockSpec((BN, D_LOCAL), lambda i: (blk0 + i, 0)),
        pl.BlockSpec((D_IN,), lambda i: (0,)),
        pl.BlockSpec((D_IN,), lambda i: (0,)),
        pl.BlockSpec((D_IN, D_HID), lambda i: (0, 0)),
        pl.BlockSpec((D_HID,), lambda i: (0,)),
        pl.BlockSpec((D_HID, D_OUT), lambda i: (0, 0)),
        pl.BlockSpec((D_OUT,), lambda i: (0,)),
    ]
    out_spec = pl.BlockSpec((BN, D_OUT), lambda i: (blk0 + i, 0))
    return in_specs, out_spec


def _tc_mlp_part(part, pooled, local_feats, ln_scale, ln_bias,
                 W1, b1, W2, b2, carry):
    blk0 = part * BLK_PER_PART
    nblk = min(N // BN - blk0, BLK_PER_PART)
    in_specs, out_spec = _mk_specs(blk0)
    args = (pooled, local_feats, ln_scale, ln_bias, W1, b1, W2, b2)
    if part == 0:
        body = _tc_mlp_body
        aliases = {}
    else:
        body = _tc_mlp_chain_body
        in_specs = in_specs + [pl.BlockSpec(memory_space=pl.ANY)]
        args = args + (carry,)
        aliases = {8: 0}
    return pl.pallas_call(
        body,
        grid=(nblk,),
        in_specs=in_specs,
        out_specs=out_spec,
        out_shape=jax.ShapeDtypeStruct((N, D_OUT), jnp.float32),
        input_output_aliases=aliases,
        compiler_params=pltpu.CompilerParams(
            dimension_semantics=("arbitrary",),
        ),
    )(*args)


def kernel(anchor_feats, local_feats, anchor_idx_per_point, dists,
           log_temp, ln_scale, ln_bias, W1, b1, W2, b2):
    idxf = jnp.pad(anchor_idx_per_point.astype(jnp.int32),
                   ((0, NP - N), (0, 0))).reshape(-1)
    idxf = jnp.concatenate([idxf, jnp.zeros((SLACK,), jnp.int32)])
    distf = jnp.pad(dists, ((0, NP - N), (0, 0))).reshape(-1)
    temp = jnp.clip(jnp.exp(log_temp), 0.5, 200.0).astype(jnp.float32)
    tempv = jnp.broadcast_to(temp, (128,))

    pooled_parts = []
    for i in range(H):
        s = i * PART * K
        idx_i = lax.slice(idxf, (s,), (s + PART * K + SLACK,))
        dist_i = lax.slice(distf, (s,), (s + PART * K,))
        pooled_parts.append(_sc_pool(anchor_feats, idx_i, dist_i, tempv))

    W1b = W1.astype(jnp.bfloat16)
    W2b = W2.astype(jnp.bfloat16)
    out = None
    for i in range(H):
        out = _tc_mlp_part(i, pooled_parts[i], local_feats, ln_scale, ln_bias,
                           W1b, b1, W2b, b2, out)
    return out
